# baseline (speedup 1.0000x reference)
import functools

import jax
import jax.numpy as jnp
from jax.experimental import pallas as pl
from jax.experimental.pallas import tpu as pltpu

HIDDEN = 4096
NUM_EXPERTS = 64
TOP_K = 8
TOKENS = 16384
BLOCK = 256


def _router_body(x_ref, wt_ref, scores_ref, w_ref, idx_ref):
    logits = jax.lax.dot_general(
        x_ref[...].astype(jnp.bfloat16), wt_ref[...].astype(jnp.bfloat16),
        dimension_numbers=(((1,), (0,)), ((), ())),
        preferred_element_type=jnp.float32,
    )
    m = jnp.max(logits, axis=1, keepdims=True)
    e = jnp.exp(logits - m)
    s = jnp.sum(e, axis=1, keepdims=True)
    scores = e / s
    scores_ref[...] = scores

    iota = jax.lax.broadcasted_iota(jnp.int32, scores.shape, 1)
    cur = scores
    vals = []
    idxs = []
    for _ in range(TOP_K):
        mx = jnp.max(cur, axis=1, keepdims=True)
        amx = jnp.min(jnp.where(cur == mx, iota, NUM_EXPERTS),
                      axis=1, keepdims=True)
        vals.append(mx)
        idxs.append(amx)
        cur = jnp.where(iota == amx, -1.0, cur)
    v = jnp.concatenate(vals, axis=1)
    ii = jnp.concatenate(idxs, axis=1)
    norm = jnp.sum(v, axis=1, keepdims=True)
    w_ref[...] = v / norm
    idx_ref[...] = ii


@functools.partial(jax.jit, static_argnames=())
def kernel(x, W):
    wt = W.T
    grid = (TOKENS // BLOCK,)
    scores, weights, top_experts = pl.pallas_call(
        _router_body,
        grid=grid,
        in_specs=[
            pl.BlockSpec((BLOCK, HIDDEN), lambda i: (i, 0)),
            pl.BlockSpec((HIDDEN, NUM_EXPERTS), lambda i: (0, 0)),
        ],
        out_specs=[
            pl.BlockSpec((BLOCK, NUM_EXPERTS), lambda i: (i, 0)),
            pl.BlockSpec((BLOCK, TOP_K), lambda i: (i, 0)),
            pl.BlockSpec((BLOCK, TOP_K), lambda i: (i, 0)),
        ],
        out_shape=[
            jax.ShapeDtypeStruct((TOKENS, NUM_EXPERTS), jnp.float32),
            jax.ShapeDtypeStruct((TOKENS, TOP_K), jnp.float32),
            jax.ShapeDtypeStruct((TOKENS, TOP_K), jnp.int32),
        ],
        compiler_params=pltpu.CompilerParams(
            dimension_semantics=("arbitrary",),
        ),
    )(x, wt)
    return (scores, weights, top_experts)


# speedup vs baseline: 1.4403x; 1.4403x over previous
import jax
import jax.numpy as jnp
from jax.experimental import pallas as pl
from jax.experimental.pallas import tpu as pltpu

HIDDEN = 4096
NUM_EXPERTS = 64
TOP_K = 8
TOKENS = 16384
BLOCK = 256


def _router_body(x_ref, wt_ref, scores_ref, w_ref, idx_ref):
    logits = jax.lax.dot_general(
        x_ref[...].astype(jnp.bfloat16), wt_ref[...].astype(jnp.bfloat16),
        dimension_numbers=(((1,), (0,)), ((), ())),
        preferred_element_type=jnp.float32,
    )
    lt = logits.T
    m = jnp.max(lt, axis=0, keepdims=True)
    e = jnp.exp(lt - m)
    s = jnp.sum(e, axis=0, keepdims=True)
    scores_t = e / s
    scores_ref[...] = scores_t.T

    iota = jax.lax.broadcasted_iota(jnp.int32, scores_t.shape, 0)
    cur = scores_t
    vals = []
    idxs = []
    for _ in range(TOP_K):
        mx = jnp.max(cur, axis=0, keepdims=True)
        amx = jnp.min(jnp.where(cur == mx, iota, NUM_EXPERTS),
                      axis=0, keepdims=True)
        vals.append(mx)
        idxs.append(amx)
        cur = jnp.where(iota == amx, -1.0, cur)
    v = jnp.concatenate(vals, axis=0)
    ii = jnp.concatenate(idxs, axis=0)
    norm = jnp.sum(v, axis=0, keepdims=True)
    w_ref[...] = (v / norm).T
    idx_ref[...] = ii.T


def kernel(x, W):
    wt = W.T
    grid = (TOKENS // BLOCK,)
    scores, weights, top_experts = pl.pallas_call(
        _router_body,
        grid=grid,
        in_specs=[
            pl.BlockSpec((BLOCK, HIDDEN), lambda i: (i, 0)),
            pl.BlockSpec((HIDDEN, NUM_EXPERTS), lambda i: (0, 0)),
        ],
        out_specs=[
            pl.BlockSpec((BLOCK, NUM_EXPERTS), lambda i: (i, 0)),
            pl.BlockSpec((BLOCK, TOP_K), lambda i: (i, 0)),
            pl.BlockSpec((BLOCK, TOP_K), lambda i: (i, 0)),
        ],
        out_shape=[
            jax.ShapeDtypeStruct((TOKENS, NUM_EXPERTS), jnp.float32),
            jax.ShapeDtypeStruct((TOKENS, TOP_K), jnp.float32),
            jax.ShapeDtypeStruct((TOKENS, TOP_K), jnp.int32),
        ],
        compiler_params=pltpu.CompilerParams(
            dimension_semantics=("arbitrary",),
        ),
    )(x, wt)
    return (scores, weights, top_experts)


# speedup vs baseline: 1.7183x; 1.1931x over previous
import jax
import jax.numpy as jnp
from jax.experimental import pallas as pl
from jax.experimental.pallas import tpu as pltpu

HIDDEN = 4096
NUM_EXPERTS = 64
TOP_K = 8
TOKENS = 16384
BLOCK = 512


def _router_body(x_ref, wt_ref, scores_ref, w_ref, idx_ref):
    logits = jax.lax.dot_general(
        x_ref[...].astype(jnp.bfloat16), wt_ref[...].astype(jnp.bfloat16),
        dimension_numbers=(((1,), (0,)), ((), ())),
        preferred_element_type=jnp.float32,
    )
    lt = logits.T
    m = jnp.max(lt, axis=0, keepdims=True)
    e = jnp.exp(lt - m)
    s = jnp.sum(e, axis=0, keepdims=True)
    scores_t = e / s
    scores_ref[...] = scores_t.T

    iota = jax.lax.broadcasted_iota(jnp.int32, scores_t.shape, 0)
    cur = scores_t
    vals = []
    idxs = []
    for _ in range(TOP_K):
        mx = jnp.max(cur, axis=0, keepdims=True)
        amx = jnp.min(jnp.where(cur == mx, iota, NUM_EXPERTS),
                      axis=0, keepdims=True)
        vals.append(mx)
        idxs.append(amx)
        cur = jnp.where(iota == amx, -1.0, cur)
    v = jnp.concatenate(vals, axis=0)
    ii = jnp.concatenate(idxs, axis=0)
    norm = jnp.sum(v, axis=0, keepdims=True)
    w_ref[...] = (v / norm).T
    idx_ref[...] = ii.T


def kernel(x, W):
    wt = W.T
    grid = (TOKENS // BLOCK,)
    scores, weights, top_experts = pl.pallas_call(
        _router_body,
        grid=grid,
        in_specs=[
            pl.BlockSpec((BLOCK, HIDDEN), lambda i: (i, 0)),
            pl.BlockSpec((HIDDEN, NUM_EXPERTS), lambda i: (0, 0)),
        ],
        out_specs=[
            pl.BlockSpec((BLOCK, NUM_EXPERTS), lambda i: (i, 0)),
            pl.BlockSpec((BLOCK, TOP_K), lambda i: (i, 0)),
            pl.BlockSpec((BLOCK, TOP_K), lambda i: (i, 0)),
        ],
        out_shape=[
            jax.ShapeDtypeStruct((TOKENS, NUM_EXPERTS), jnp.float32),
            jax.ShapeDtypeStruct((TOKENS, TOP_K), jnp.float32),
            jax.ShapeDtypeStruct((TOKENS, TOP_K), jnp.int32),
        ],
        compiler_params=pltpu.CompilerParams(
            dimension_semantics=("arbitrary",),
        ),
    )(x, wt)
    return (scores, weights, top_experts)


# speedup vs baseline: 1.8061x; 1.0511x over previous
import jax
import jax.numpy as jnp
from jax.experimental import pallas as pl
from jax.experimental.pallas import tpu as pltpu

HIDDEN = 4096
NUM_EXPERTS = 64
TOP_K = 8
TOKENS = 16384
BLOCK = 1024


def _router_body(x_ref, wt_ref, scores_ref, w_ref, idx_ref):
    logits = jax.lax.dot_general(
        x_ref[...].astype(jnp.bfloat16), wt_ref[...].astype(jnp.bfloat16),
        dimension_numbers=(((1,), (0,)), ((), ())),
        preferred_element_type=jnp.float32,
    )
    lt = logits.T
    m = jnp.max(lt, axis=0, keepdims=True)
    e = jnp.exp(lt - m)
    s = jnp.sum(e, axis=0, keepdims=True)
    scores_t = e / s
    scores_ref[...] = scores_t.T

    iota = jax.lax.broadcasted_iota(jnp.int32, scores_t.shape, 0)
    cur = scores_t
    vals = []
    idxs = []
    for _ in range(TOP_K):
        mx = jnp.max(cur, axis=0, keepdims=True)
        amx = jnp.min(jnp.where(cur == mx, iota, NUM_EXPERTS),
                      axis=0, keepdims=True)
        vals.append(mx)
        idxs.append(amx)
        cur = jnp.where(iota == amx, -1.0, cur)
    v = jnp.concatenate(vals, axis=0)
    ii = jnp.concatenate(idxs, axis=0)
    norm = jnp.sum(v, axis=0, keepdims=True)
    w_ref[...] = (v / norm).T
    idx_ref[...] = ii.T


def kernel(x, W):
    wt = W.T
    grid = (TOKENS // BLOCK,)
    scores, weights, top_experts = pl.pallas_call(
        _router_body,
        grid=grid,
        in_specs=[
            pl.BlockSpec((BLOCK, HIDDEN), lambda i: (i, 0)),
            pl.BlockSpec((HIDDEN, NUM_EXPERTS), lambda i: (0, 0)),
        ],
        out_specs=[
            pl.BlockSpec((BLOCK, NUM_EXPERTS), lambda i: (i, 0)),
            pl.BlockSpec((BLOCK, TOP_K), lambda i: (i, 0)),
            pl.BlockSpec((BLOCK, TOP_K), lambda i: (i, 0)),
        ],
        out_shape=[
            jax.ShapeDtypeStruct((TOKENS, NUM_EXPERTS), jnp.float32),
            jax.ShapeDtypeStruct((TOKENS, TOP_K), jnp.float32),
            jax.ShapeDtypeStruct((TOKENS, TOP_K), jnp.int32),
        ],
        compiler_params=pltpu.CompilerParams(
            dimension_semantics=("arbitrary",),
        ),
    )(x, wt)
    return (scores, weights, top_experts)


# speedup vs baseline: 1.8088x; 1.0015x over previous
import jax
import jax.numpy as jnp
from jax.experimental import pallas as pl
from jax.experimental.pallas import tpu as pltpu

HIDDEN = 4096
NUM_EXPERTS = 64
TOP_K = 8
TOKENS = 16384
BLOCK = 1024


def _router_body(x_ref, wt_ref, scores_ref, w_ref, idx_ref):
    logits = jax.lax.dot_general(
        x_ref[...].astype(jnp.bfloat16), wt_ref[...].astype(jnp.bfloat16),
        dimension_numbers=(((1,), (0,)), ((), ())),
        preferred_element_type=jnp.float32,
    )
    lt = logits.T
    m = jnp.max(lt, axis=0, keepdims=True)
    e = jnp.exp(lt - m)
    s = jnp.sum(e, axis=0, keepdims=True)
    scores_t = e / s
    scores_ref[...] = scores_t.T

    iota = jax.lax.broadcasted_iota(jnp.int32, scores_t.shape, 0)
    cur = scores_t
    vals = []
    idxs = []
    for _ in range(TOP_K):
        mx = jnp.max(cur, axis=0, keepdims=True)
        amx = jnp.min(jnp.where(cur == mx, iota, NUM_EXPERTS),
                      axis=0, keepdims=True)
        vals.append(mx)
        idxs.append(amx)
        cur = jnp.where(iota == amx, -1.0, cur)
    v = jnp.concatenate(vals, axis=0)
    ii = jnp.concatenate(idxs, axis=0)
    norm = jnp.sum(v, axis=0, keepdims=True)
    w_ref[...] = (v / norm).T
    idx_ref[...] = ii.T


def kernel(x, W):
    wt = W.T
    grid = (TOKENS // BLOCK,)
    scores, weights, top_experts = pl.pallas_call(
        _router_body,
        grid=grid,
        in_specs=[
            pl.BlockSpec((BLOCK, HIDDEN), lambda i: (i, 0)),
            pl.BlockSpec((HIDDEN, NUM_EXPERTS), lambda i: (0, 0)),
        ],
        out_specs=[
            pl.BlockSpec((BLOCK, NUM_EXPERTS), lambda i: (i, 0)),
            pl.BlockSpec((BLOCK, TOP_K), lambda i: (i, 0)),
            pl.BlockSpec((BLOCK, TOP_K), lambda i: (i, 0)),
        ],
        out_shape=[
            jax.ShapeDtypeStruct((TOKENS, NUM_EXPERTS), jnp.float32),
            jax.ShapeDtypeStruct((TOKENS, TOP_K), jnp.float32),
            jax.ShapeDtypeStruct((TOKENS, TOP_K), jnp.int32),
        ],
        compiler_params=pltpu.CompilerParams(
            dimension_semantics=("parallel",),
        ),
    )(x, wt)
    return (scores, weights, top_experts)
